# Initial kernel scaffold; baseline (speedup 1.0000x reference)
import jax
import jax.numpy as jnp
from jax.experimental import pallas as pl


def kernel(s_idx, t_idx, neg_idx, hist_s, hist_t, hist_dist, mp_type, neg_hist_s, neg_hist_t, neg_hist_dist, neg_mp_type, s_time, hist_time, neg_hist_time, emb, delta, distance_att, metapath_att):
    raise NotImplementedError("write your pallas kernel here")


# speedup vs baseline: 1.1455x; 1.1455x over previous
import functools

import jax
import jax.numpy as jnp
from jax import lax
from jax.experimental import pallas as pl
from jax.experimental.pallas import tpu as pltpu
from jax.experimental.pallas import tpu_sc as plsc

_B = 1024
_D = 128
_M = 3
_H = 8
_Q = 5
_CE = 5
_MT = 3

_NC = 2
_NS = 16
_NW = _NC * _NS
_L = 16

_P = 152
_PW = _B * _P // _NW
_CH = 128
_NCHUNK = _PW // _CH
_ND = 2 * _B + _B * _Q
_DW = _ND // _NW
_DC = 112


def _sc_sims(emb, u_flat, v_flat, didx, delta):
    mesh = plsc.VectorSubcoreMesh(core_axis_name="c", subcore_axis_name="s")

    @functools.partial(
        pl.kernel,
        mesh=mesh,
        compiler_params=pltpu.CompilerParams(needs_layout_passes=False),
        out_type=[
            jax.ShapeDtypeStruct((_B * _P,), jnp.float32),
            jax.ShapeDtypeStruct((_ND,), jnp.float32),
        ],
        scratch_types=[
            pltpu.VMEM((_CH,), jnp.int32),
            pltpu.VMEM((_CH,), jnp.int32),
            pltpu.VMEM((_CH, _D), jnp.float32),
            pltpu.VMEM((_CH, _D), jnp.float32),
            pltpu.VMEM((_CH,), jnp.float32),
            pltpu.VMEM((_L * _L,), jnp.float32),
            pltpu.VMEM((_DC,), jnp.int32),
            pltpu.VMEM((_DC,), jnp.float32),
            pltpu.SemaphoreType.DMA,
        ],
    )
    def k(emb_h, u_h, v_h, didx_h, delta_h, sims_h, dval_h,
          uix, vix, urows, vrows, outv, tbuf, dix, dvv, sem):
        wid = lax.axis_index("s") * _NC + lax.axis_index("c")
        pbase = wid * _PW
        lane = lax.iota(jnp.int32, _L)

        def chunk_body(g, carry):
            off = pbase + g * _CH
            pltpu.sync_copy(u_h.at[pl.ds(off, _CH)], uix)
            pltpu.sync_copy(v_h.at[pl.ds(off, _CH)], vix)
            cu = pltpu.async_copy(emb_h.at[uix], urows, sem)
            cv = pltpu.async_copy(emb_h.at[vix], vrows, sem)
            cu.wait()
            cv.wait()

            def pg_body(pgi, carry2):
                for i in range(_L):
                    p = pgi * _L + i
                    acc = jnp.zeros((_L,), jnp.float32)
                    for kk in range(_D // _L):
                        uu = urows[p, pl.ds(kk * _L, _L)]
                        vv = vrows[p, pl.ds(kk * _L, _L)]
                        df = uu - vv
                        acc = acc + df * df
                    plsc.store_scatter(tbuf, [lane * _L + i], acc)
                tot = jnp.zeros((_L,), jnp.float32)
                for j in range(_L):
                    tot = tot + tbuf[pl.ds(j * _L, _L)]
                outv[pl.ds(pgi * _L, _L)] = -tot
                return carry2

            lax.fori_loop(0, _CH // _L, pg_body, 0)
            pltpu.sync_copy(outv, sims_h.at[pl.ds(off, _CH)])
            return carry

        lax.fori_loop(0, _NCHUNK, chunk_body, 0)

        dbase = wid * _DW

        def d_chunk(h, carry):
            doff = dbase + h * _DC
            pltpu.sync_copy(didx_h.at[pl.ds(doff, _DC)], dix)
            pltpu.async_copy(delta_h.at[dix], dvv, sem).wait()
            pltpu.sync_copy(dvv, dval_h.at[pl.ds(doff, _DC)])
            return carry

        lax.fori_loop(0, _DW // _DC, d_chunk, 0)

    return k(emb, u_flat, v_flat, didx, delta)


def _combine_body(st_r, hs_r, ns_r, nh_r,
                  hdist_r, htime_r, hmp_r,
                  ndist_r, ntime_r, nmp_r,
                  stime_r, ds_r, dt_r, dneg_r,
                  wd_r, wm_r, out_r):
    def softmax_row(row):
        mx = jnp.max(row, axis=1, keepdims=True)
        e = jnp.exp(row - mx)
        return e / jnp.sum(e, axis=1, keepdims=True)

    wd = softmax_row(wd_r[:])
    wm = softmax_row(wm_r[:])

    def takew(w, idx, k):
        acc = jnp.zeros(idx.shape, jnp.float32)
        for c in range(k):
            acc = acc + jnp.where(idx == c, 1.0, 0.0) * w[:, c:c + 1]
        return acc

    stime = stime_r[:]
    ds = ds_r[:]

    def branch(sim, dist, time, mp, dprod):
        dtt = jnp.abs(stime - time)
        decay = jnp.exp(-dprod * dtt)
        single = sim * takew(wd, dist, _CE) * decay * takew(wm, mp, _MT)
        return jnp.sum(single, axis=1, keepdims=True)

    p_lam = st_r[:] + branch(hs_r[:], hdist_r[:], htime_r[:], hmp_r[:],
                             ds * dt_r[:])

    nh = nh_r[:]
    ndist = ndist_r[:]
    ntime = ntime_r[:]
    nmp = nmp_r[:]
    ns = ns_r[:]
    dneg = dneg_r[:]
    seg = _M * _H
    nacc = jnp.zeros((_B, 1), jnp.float32)
    for q in range(_Q):
        lo = q * seg
        seg_sum = branch(nh[:, lo:lo + seg], ndist[:, lo:lo + seg],
                         ntime[:, lo:lo + seg], nmp[:, lo:lo + seg],
                         ds * dneg[:, q:q + 1])
        nlam = ns[:, q:q + 1] + seg_sum
        sig = 1.0 / (1.0 + jnp.exp(nlam))
        nacc = nacc + jnp.log(sig + 1e-06)
    sigp = 1.0 / (1.0 + jnp.exp(-p_lam))
    out_r[:] = -jnp.log(sigp + 1e-06) - nacc


def kernel(s_idx, t_idx, neg_idx, hist_s, hist_t, hist_dist, mp_type,
           neg_hist_s, neg_hist_t, neg_hist_dist, neg_mp_type,
           s_time, hist_time, neg_hist_time,
           emb, delta, distance_att, metapath_att):
    i32 = jnp.int32
    pad = jnp.zeros((_B, _P - 150), i32)
    u_flat = jnp.concatenate([
        s_idx[:, None].astype(i32),
        hist_s.reshape(_B, _M * _H).astype(i32),
        jnp.broadcast_to(s_idx[:, None].astype(i32), (_B, _Q)),
        neg_hist_s.reshape(_B, _Q * _M * _H).astype(i32),
        pad,
    ], axis=1).reshape(-1)
    v_flat = jnp.concatenate([
        t_idx[:, None].astype(i32),
        hist_t.reshape(_B, _M * _H).astype(i32),
        neg_idx.astype(i32),
        neg_hist_t.reshape(_B, _Q * _M * _H).astype(i32),
        pad,
    ], axis=1).reshape(-1)
    didx = jnp.concatenate([s_idx.astype(i32), t_idx.astype(i32),
                            neg_idx.astype(i32).reshape(-1)])

    sims_flat, dvals = _sc_sims(emb, u_flat, v_flat, didx, delta)

    sims = sims_flat.reshape(_B, _P)
    st = sims[:, 0:1]
    hs = sims[:, 1:1 + _M * _H]
    ns = sims[:, 25:25 + _Q]
    nh = sims[:, 30:30 + _Q * _M * _H]

    ds = dvals[:_B][:, None]
    dt = dvals[_B:2 * _B][:, None]
    dneg = dvals[2 * _B:].reshape(_B, _Q)

    hdist = hist_dist.reshape(_B, _M * _H).astype(i32)
    htime = hist_time.reshape(_B, _M * _H)
    hmp = jnp.repeat(mp_type.astype(i32), _H, axis=1)
    ndist = neg_hist_dist.reshape(_B, _Q * _M * _H).astype(i32)
    ntime = neg_hist_time.reshape(_B, _Q * _M * _H)
    nmp = jnp.repeat(neg_mp_type.astype(i32).reshape(_B, _Q * _M), _H, axis=1)
    stime = s_time[:, None]

    neg_inf = jnp.float32(-1e30)
    wd_row = jnp.full((1, 128), neg_inf).at[0, :_CE].set(distance_att)
    wm_row = jnp.full((1, 128), neg_inf).at[0, :_MT].set(metapath_att)

    loss2 = pl.pallas_call(
        _combine_body,
        out_shape=jax.ShapeDtypeStruct((_B, 1), jnp.float32),
    )(st, hs, ns, nh, hdist, htime, hmp, ndist, ntime, nmp,
      stime, ds, dt, dneg, wd_row, wm_row)
    return loss2[:, 0]


# speedup vs baseline: 1.1790x; 1.0293x over previous
import functools

import jax
import jax.numpy as jnp
from jax import lax
from jax.experimental import pallas as pl
from jax.experimental.pallas import tpu as pltpu
from jax.experimental.pallas import tpu_sc as plsc

_B = 1024
_D = 128
_M = 3
_H = 8
_Q = 5
_CE = 5
_MT = 3

_NC = 2
_NS = 16
_NW = _NC * _NS
_L = 16

_P = 152
_PW = _B * _P // _NW
_CH = 128
_NCHUNK = _PW // _CH
_ND = 2 * _B + _B * _Q
_DW = _ND // _NW
_DC = 112


def _sc_sims(emb, u2, v2, didx2, delta):
    mesh = plsc.VectorSubcoreMesh(core_axis_name="c", subcore_axis_name="s")

    @functools.partial(
        pl.kernel,
        mesh=mesh,
        compiler_params=pltpu.CompilerParams(needs_layout_passes=False),
        out_type=[
            jax.ShapeDtypeStruct((_B * _P,), jnp.float32),
            jax.ShapeDtypeStruct((_ND,), jnp.float32),
        ],
        scratch_types=[
            pltpu.VMEM((_PW,), jnp.int32),
            pltpu.VMEM((_PW,), jnp.int32),
            pltpu.VMEM((_CH, _D), jnp.float32),
            pltpu.VMEM((_CH, _D), jnp.float32),
            pltpu.VMEM((_CH, _D), jnp.float32),
            pltpu.VMEM((_CH, _D), jnp.float32),
            pltpu.VMEM((_PW,), jnp.float32),
            pltpu.VMEM((_L * _L,), jnp.float32),
            pltpu.VMEM((_DW,), jnp.int32),
            pltpu.VMEM((_DW,), jnp.float32),
            pltpu.SemaphoreType.DMA,
            pltpu.SemaphoreType.DMA,
        ],
    )
    def k(emb_h, u_h, v_h, didx_h, delta_h, sims_h, dval_h,
          uix, vix, ur0, vr0, ur1, vr1, outv, tbuf, dix, dvv, sem0, sem1):
        urows_ = (ur0, ur1)
        vrows_ = (vr0, vr1)
        sems = (sem0, sem1)
        wid = lax.axis_index("s") * _NC + lax.axis_index("c")
        pbase = wid * _PW
        lane = lax.iota(jnp.int32, _L)

        pltpu.sync_copy(u_h.at[pl.ds(pbase, _PW)], uix)
        pltpu.sync_copy(v_h.at[pl.ds(pbase, _PW)], vix)

        def fire(b, g):
            off = g * _CH
            pltpu.async_copy(emb_h.at[uix.at[pl.ds(off, _CH)]], urows_[b], sems[b])
            pltpu.async_copy(emb_h.at[vix.at[pl.ds(off, _CH)]], vrows_[b], sems[b])

        def wait(b):
            pltpu.make_async_copy(emb_h.at[uix.at[pl.ds(0, _CH)]], urows_[b], sems[b]).wait()
            pltpu.make_async_copy(emb_h.at[vix.at[pl.ds(0, _CH)]], vrows_[b], sems[b]).wait()

        def compute(b, g):
            urows = urows_[b]
            vrows = vrows_[b]

            def pg_body(pgi, carry2):
                for i in range(_L):
                    p = pgi * _L + i
                    acc = jnp.zeros((_L,), jnp.float32)
                    for kk in range(_D // _L):
                        uu = urows[p, pl.ds(kk * _L, _L)]
                        vv = vrows[p, pl.ds(kk * _L, _L)]
                        df = uu - vv
                        acc = acc + df * df
                    plsc.store_scatter(tbuf, [lane * _L + i], acc)
                tot = jnp.zeros((_L,), jnp.float32)
                for j in range(_L):
                    tot = tot + tbuf[pl.ds(j * _L, _L)]
                outv[pl.ds(g * _CH + pgi * _L, _L)] = -tot
                return carry2

            lax.fori_loop(0, _CH // _L, pg_body, 0)

        fire(0, 0)

        def outer(i, carry):
            for b in range(2):
                g = 2 * i + b

                @pl.when(g + 1 < _NCHUNK)
                def _():
                    fire(1 - b, g + 1)

                wait(b)
                compute(b, g)
            return carry

        lax.fori_loop(0, _NCHUNK // 2, outer, 0)
        pltpu.sync_copy(outv, sims_h.at[pl.ds(pbase, _PW)])

        dbase = wid * _DW
        pltpu.sync_copy(didx_h.at[pl.ds(dbase, _DW)], dix)
        for h in range(_DW // _DC):
            pltpu.async_copy(delta_h.at[dix.at[pl.ds(h * _DC, _DC)]],
                             dvv.at[pl.ds(h * _DC, _DC)], sem0).wait()
        pltpu.sync_copy(dvv, dval_h.at[pl.ds(dbase, _DW)])

    return k(emb, u2, v2, didx2, delta)


def _combine_body(st_r, hs_r, ns_r, nh_r,
                  hdist_r, htime_r, hmp_r,
                  ndist_r, ntime_r, nmp_r,
                  stime_r, ds_r, dt_r, dneg_r,
                  wd_r, wm_r, out_r):
    def softmax_row(row):
        mx = jnp.max(row, axis=1, keepdims=True)
        e = jnp.exp(row - mx)
        return e / jnp.sum(e, axis=1, keepdims=True)

    wd = softmax_row(wd_r[:])
    wm = softmax_row(wm_r[:])

    def takew(w, idx, k):
        acc = jnp.zeros(idx.shape, jnp.float32)
        for c in range(k):
            acc = acc + jnp.where(idx == c, 1.0, 0.0) * w[:, c:c + 1]
        return acc

    stime = stime_r[:]
    ds = ds_r[:]

    def branch(sim, dist, time, mp, dprod):
        dtt = jnp.abs(stime - time)
        decay = jnp.exp(-dprod * dtt)
        single = sim * takew(wd, dist, _CE) * decay * takew(wm, mp, _MT)
        return jnp.sum(single, axis=1, keepdims=True)

    p_lam = st_r[:] + branch(hs_r[:], hdist_r[:], htime_r[:], hmp_r[:],
                             ds * dt_r[:])

    nh = nh_r[:]
    ndist = ndist_r[:]
    ntime = ntime_r[:]
    nmp = nmp_r[:]
    ns = ns_r[:]
    dneg = dneg_r[:]
    seg = _M * _H
    nacc = jnp.zeros((_B, 1), jnp.float32)
    for q in range(_Q):
        lo = q * seg
        seg_sum = branch(nh[:, lo:lo + seg], ndist[:, lo:lo + seg],
                         ntime[:, lo:lo + seg], nmp[:, lo:lo + seg],
                         ds * dneg[:, q:q + 1])
        nlam = ns[:, q:q + 1] + seg_sum
        sig = 1.0 / (1.0 + jnp.exp(nlam))
        nacc = nacc + jnp.log(sig + 1e-06)
    sigp = 1.0 / (1.0 + jnp.exp(-p_lam))
    out_r[:] = -jnp.log(sigp + 1e-06) - nacc


def kernel(s_idx, t_idx, neg_idx, hist_s, hist_t, hist_dist, mp_type,
           neg_hist_s, neg_hist_t, neg_hist_dist, neg_mp_type,
           s_time, hist_time, neg_hist_time,
           emb, delta, distance_att, metapath_att):
    i32 = jnp.int32
    pad = jnp.zeros((_B, _P - 150), i32)
    u_flat = jnp.concatenate([
        s_idx[:, None].astype(i32),
        hist_s.reshape(_B, _M * _H).astype(i32),
        jnp.broadcast_to(s_idx[:, None].astype(i32), (_B, _Q)),
        neg_hist_s.reshape(_B, _Q * _M * _H).astype(i32),
        pad,
    ], axis=1).reshape(-1)
    v_flat = jnp.concatenate([
        t_idx[:, None].astype(i32),
        hist_t.reshape(_B, _M * _H).astype(i32),
        neg_idx.astype(i32),
        neg_hist_t.reshape(_B, _Q * _M * _H).astype(i32),
        pad,
    ], axis=1).reshape(-1)
    didx = jnp.concatenate([s_idx.astype(i32), t_idx.astype(i32),
                            neg_idx.astype(i32).reshape(-1)])

    sims_flat, dvals = _sc_sims(emb, u_flat, v_flat, didx, delta)
    sims = sims_flat.reshape(_B, _P)
    st = sims[:, 0:1]
    hs = sims[:, 1:1 + _M * _H]
    ns = sims[:, 25:25 + _Q]
    nh = sims[:, 30:30 + _Q * _M * _H]

    ds = dvals[:_B][:, None]
    dt = dvals[_B:2 * _B][:, None]
    dneg = dvals[2 * _B:].reshape(_B, _Q)

    hdist = hist_dist.reshape(_B, _M * _H).astype(i32)
    htime = hist_time.reshape(_B, _M * _H)
    hmp = jnp.repeat(mp_type.astype(i32), _H, axis=1)
    ndist = neg_hist_dist.reshape(_B, _Q * _M * _H).astype(i32)
    ntime = neg_hist_time.reshape(_B, _Q * _M * _H)
    nmp = jnp.repeat(neg_mp_type.astype(i32).reshape(_B, _Q * _M), _H, axis=1)
    stime = s_time[:, None]

    neg_inf = jnp.float32(-1e30)
    wd_row = jnp.full((1, 128), neg_inf).at[0, :_CE].set(distance_att)
    wm_row = jnp.full((1, 128), neg_inf).at[0, :_MT].set(metapath_att)

    loss2 = pl.pallas_call(
        _combine_body,
        out_shape=jax.ShapeDtypeStruct((_B, 1), jnp.float32),
    )(st, hs, ns, nh, hdist, htime, hmp, ndist, ntime, nmp,
      stime, ds, dt, dneg, wd_row, wm_row)
    return loss2[:, 0]
